# baseline (speedup 1.0000x reference)
import functools
import jax
import jax.numpy as jnp
from jax import lax
from jax.experimental import pallas as pl
from jax.experimental.pallas import tpu as pltpu
from jax.experimental.pallas import tpu_sc as plsc

N = 10000
E = 320000
D = 128
DE = 16

NC = 2
NS = 16
NW = NC * NS
EPT = E // NW
CH = 50
NCHK = EPT // CH
NBUF = 4
L = 16
NPT = N // NS

_MESH = plsc.VectorSubcoreMesh(
    core_axis_name="c", subcore_axis_name="s", num_cores=NC, num_subcores=NS)

f32 = jnp.float32
i32 = jnp.int32


def _node_mm(x, w, att, relu_in=False):
    B = 1000

    def body(x_ref, w_ref, att_ref, h_ref, a_ref, m_ref):
        xb = x_ref[...]
        if relu_in:
            xb = jax.nn.relu(xb)
        h = jnp.dot(xb, w_ref[...], preferred_element_type=f32)
        h_ref[...] = h
        a = jnp.dot(h, att_ref[...], preferred_element_type=f32)
        a_ref[...] = a
        m_ref[0, 0, :] = jnp.max(a, axis=0)

    return pl.pallas_call(
        body,
        grid=(N // B,),
        in_specs=[
            pl.BlockSpec((B, D), lambda i: (i, 0)),
            pl.BlockSpec((D, D), lambda i: (0, 0)),
            pl.BlockSpec((D, 2), lambda i: (0, 0)),
        ],
        out_specs=[
            pl.BlockSpec((B, D), lambda i: (i, 0)),
            pl.BlockSpec((B, 2), lambda i: (i, 0)),
            pl.BlockSpec((1, 1, 2), lambda i: (i, 0, 0)),
        ],
        out_shape=[
            jax.ShapeDtypeStruct((N, D), f32),
            jax.ShapeDtypeStruct((N, 2), f32),
            jax.ShapeDtypeStruct((N // B, 1, 2), f32),
        ],
    )(x, w, att)


def _edge_mv_body(fl_ref, we1_ref, ate1_ref, we2_ref, ate2_ref,
                  ae1_ref, ae2_ref, m_ref):
    we1 = jnp.dot(we1_ref[...], ate1_ref[...], preferred_element_type=f32)
    we2 = jnp.dot(we2_ref[...], ate2_ref[...], preferred_element_type=f32)
    ae1 = jnp.dot(fl_ref[...], we1, preferred_element_type=f32)[:, 0]
    ae2 = jnp.dot(fl_ref[...], we2, preferred_element_type=f32)[:, 0]
    ae1_ref[0, 0, :] = ae1
    ae2_ref[0, 0, :] = ae2
    m_ref[0, 0, :] = jnp.stack([jnp.max(ae1), jnp.max(ae2)])


def _edge_mv(flepe, w_edge1, att_edge1, w_edge2, att_edge2):
    B = 2000
    return pl.pallas_call(
        _edge_mv_body,
        grid=(E // B,),
        in_specs=[
            pl.BlockSpec((B, DE), lambda i: (i, 0)),
            pl.BlockSpec((DE, D), lambda i: (0, 0)),
            pl.BlockSpec((D, 1), lambda i: (0, 0)),
            pl.BlockSpec((DE, D), lambda i: (0, 0)),
            pl.BlockSpec((D, 1), lambda i: (0, 0)),
        ],
        out_specs=[
            pl.BlockSpec((1, 1, B), lambda i: (i, 0, 0)),
            pl.BlockSpec((1, 1, B), lambda i: (i, 0, 0)),
            pl.BlockSpec((1, 1, 2), lambda i: (i, 0, 0)),
        ],
        out_shape=[
            jax.ShapeDtypeStruct((E // B, 1, B), f32),
            jax.ShapeDtypeStruct((E // B, 1, B), f32),
            jax.ShapeDtypeStruct((E // B, 1, 2), f32),
        ],
    )(flepe, w_edge1, att_edge1, w_edge2, att_edge2)


def _make_edge_scalar(first):
    out_type = [jax.ShapeDtypeStruct((NW, EPT), f32),
                jax.ShapeDtypeStruct((NW, N), f32),
                jax.ShapeDtypeStruct((NW, N), f32)]
    scratch = [
        pltpu.VMEM((N,), f32),
        pltpu.VMEM((N,), f32),
        pltpu.VMEM((EPT,), i32),
        pltpu.VMEM((EPT,), i32),
        pltpu.VMEM((EPT,), f32),
        pltpu.VMEM((EPT,), f32),
        pltpu.VMEM((N,), f32),
        pltpu.VMEM((N,), f32),
        pltpu.VMEM((L,), f32),
    ]
    if first:
        out_type += [jax.ShapeDtypeStruct((NW, N), f32)]
        scratch += [pltpu.VMEM((N,), f32)]

    def body(asd_hbm, src_hbm, dst_hbm, ae_hbm, c_hbm, *rest):
        if first:
            (ex_hbm, es_hbm, sae_hbm, deg_hbm,
             asv, adv, srcv, dstv, aev, exv, esv, saev, cv, degv) = rest
        else:
            (ex_hbm, es_hbm, sae_hbm,
             asv, adv, srcv, dstv, aev, exv, esv, saev, cv) = rest
            degv = None
        wid = lax.axis_index("s") * NC + lax.axis_index("c")
        pltpu.sync_copy(asd_hbm.at[0], asv)
        pltpu.sync_copy(asd_hbm.at[1], adv)
        pltpu.sync_copy(src_hbm.at[wid], srcv)
        pltpu.sync_copy(dst_hbm.at[wid], dstv)
        pltpu.sync_copy(ae_hbm.at[wid], aev)
        pltpu.sync_copy(c_hbm, cv)

        zz = jnp.zeros((L,), f32)

        def zbody(i, _):
            esv[pl.ds(i * L, L)] = zz
            saev[pl.ds(i * L, L)] = zz
            if first:
                degv[pl.ds(i * L, L)] = zz
            return 0

        lax.fori_loop(0, N // L, zbody, 0)

        cvec = cv[...]
        ones = jnp.full((L,), 1.0, f32)

        def ebody(i, _):
            sl = pl.ds(i * L, L)
            s = srcv[sl]
            d = dstv[sl]
            ae = aev[sl]
            z = plsc.load_gather(asv, [s]) + plsc.load_gather(adv, [d]) + ae
            alpha = jnp.maximum(z, 0.2 * z)
            ex = jnp.exp(alpha - cvec)
            exv[sl] = ex
            plsc.addupdate_scatter(esv, [d], ex)
            plsc.addupdate_scatter(saev, [d], ae)
            if first:
                plsc.addupdate_scatter(degv, [d], ones)
            return 0

        lax.fori_loop(0, EPT // L, ebody, 0)

        pltpu.sync_copy(exv, ex_hbm.at[wid])
        pltpu.sync_copy(esv, es_hbm.at[wid])
        pltpu.sync_copy(saev, sae_hbm.at[wid])
        if first:
            pltpu.sync_copy(degv, deg_hbm.at[wid])

    return pl.kernel(body, out_type=out_type, mesh=_MESH, scratch_types=scratch,
                     compiler_params=pltpu.CompilerParams(needs_layout_passes=False, use_tc_tiling_on_sc=False),
                     name="edge_scalar_l1" if first else "edge_scalar_l2")


_edge_scalar_l1 = _make_edge_scalar(True)
_edge_scalar_l2 = _make_edge_scalar(False)


def _row_pass_body(h_hbm, src_hbm, dst_hbm, ex_hbm, z_hbm, num_hbm,
                   srcv, dstv, *bufs):
    rows = bufs[0:NBUF]
    exb = bufs[NBUF:2 * NBUF]
    acc = bufs[2 * NBUF]
    rsem = bufs[2 * NBUF + 1:2 * NBUF + 1 + NBUF]
    esem = bufs[2 * NBUF + 1 + NBUF:2 * NBUF + 1 + 2 * NBUF]
    ssem = bufs[2 * NBUF + 1 + 2 * NBUF:2 * NBUF + 1 + 3 * NBUF]
    cid = lax.axis_index("c")
    sid = lax.axis_index("s")
    wid = sid * NC + cid
    pltpu.sync_copy(src_hbm.at[wid], srcv)
    pltpu.sync_copy(dst_hbm.at[wid], dstv)
    pltpu.sync_copy(z_hbm, acc.at[pl.ds(sid * NPT, NPT)])
    plsc.subcore_barrier()

    def start(c, b):
        pltpu.async_copy(h_hbm.at[srcv.at[c]], rows[b], rsem[b])
        pltpu.async_copy(ex_hbm.at[wid, c], exb[b], esem[b])

    def drain_scatter(b):
        pltpu.make_async_copy(h_hbm.at[pl.ds(0, CH)], rows[b], ssem[b]).wait()

    def process(c, b):
        pltpu.make_async_copy(h_hbm.at[pl.ds(0, CH)], rows[b], rsem[b]).wait()
        pltpu.make_async_copy(ex_hbm.at[0, 0], exb[b], esem[b]).wait()

        def scale(e, _):
            eidx = jnp.full((L,), 0, i32) + e
            s = plsc.load_gather(exb[b], [eidx])
            for k in range(D // L):
                sl = pl.ds(k * L, L)
                rows[b][e, sl] = rows[b][e, sl] * s
            return 0

        lax.fori_loop(0, CH, scale, 0)
        pltpu.async_copy(rows[b], acc.at[dstv.at[c]], ssem[b], add=True)

    NG = NCHK // NBUF
    start(0, 0)
    start(1, 1)
    start(2, 2)

    def gbody(g, _):
        for k in range(NBUF):
            c = NBUF * g + k
            bn = (k + 3) % NBUF
            if k == 0:
                @pl.when(g > 0)
                def _():
                    drain_scatter(bn)
                start(c + 3, bn)
            else:
                @pl.when(g < NG - 1)
                def _():
                    drain_scatter(bn)
                    start(c + 3, bn)
            process(c, k)
        return 0

    lax.fori_loop(0, NG, gbody, 0)
    for b in range(NBUF):
        drain_scatter(b)

    plsc.subcore_barrier()
    pltpu.sync_copy(acc.at[pl.ds(sid * NPT, NPT)],
                    num_hbm.at[cid, pl.ds(sid * NPT, NPT)])


_row_pass = pl.kernel(
    _row_pass_body,
    out_type=[jax.ShapeDtypeStruct((NC, N, D), f32)],
    mesh=_MESH,
    scratch_types=[
        pltpu.VMEM((NCHK, CH), i32),
        pltpu.VMEM((NCHK, CH), i32),
    ] + [pltpu.VMEM((CH, D), f32)] * NBUF
      + [pltpu.VMEM((CH,), f32)] * NBUF
      + [pltpu.VMEM_SHARED((N, D), f32)]
      + [pltpu.SemaphoreType.DMA] * (3 * NBUF),
    compiler_params=pltpu.CompilerParams(needs_layout_passes=False, use_tc_tiling_on_sc=False),
    name="row_pass",
)


def _combine_body(nump_ref, degp_ref, saep_ref, esp_ref, a_ref, h_ref, c_ref,
                  b_ref, out_ref):
    deg = jnp.sum(degp_ref[...], axis=1)
    sae = jnp.sum(saep_ref[...], axis=1)
    es = jnp.sum(esp_ref[...], axis=1)
    z = a_ref[:, 0] + a_ref[:, 1] + sae / jnp.maximum(deg, 1.0)
    alpha = jnp.maximum(z, 0.2 * z)
    selfw = jnp.exp(alpha - c_ref[0, 0])
    denom = es + selfw
    num = nump_ref[0] + nump_ref[1] + selfw[:, None] * h_ref[...]
    out_ref[...] = num / denom[:, None] + b_ref[...]


def _combine(num_p, deg_p, sae_p, es_p, a, h, c, b):
    B = 1000
    return pl.pallas_call(
        _combine_body,
        grid=(N // B,),
        in_specs=[
            pl.BlockSpec((NC, B, D), lambda i: (0, i, 0)),
            pl.BlockSpec((B, NW), lambda i: (i, 0)),
            pl.BlockSpec((B, NW), lambda i: (i, 0)),
            pl.BlockSpec((B, NW), lambda i: (i, 0)),
            pl.BlockSpec((B, 2), lambda i: (i, 0)),
            pl.BlockSpec((B, D), lambda i: (i, 0)),
            pl.BlockSpec((1, 1), lambda i: (0, 0)),
            pl.BlockSpec((1, D), lambda i: (0, 0)),
        ],
        out_specs=pl.BlockSpec((B, D), lambda i: (i, 0)),
        out_shape=jax.ShapeDtypeStruct((N, D), f32),
    )(num_p, deg_p, sae_p, es_p, a, h, c, b)


def kernel(x, edge_index, flepe, W1, att_src1, att_dst1, W_edge1, att_edge1, b1,
           W2, att_src2, att_dst2, W_edge2, att_edge2, b2):
    srcf = edge_index[0].reshape(NW, EPT)
    dstf = edge_index[1].reshape(NW, EPT)
    src3 = srcf.reshape(NW, NCHK, CH)
    dst3 = dstf.reshape(NW, NCHK, CH)
    zeros = jnp.zeros((NPT, D), f32)

    ae1_r, ae2_r, ae_m = _edge_mv(flepe, W_edge1, att_edge1[:, None],
                                  W_edge2, att_edge2[:, None])
    ae1 = ae1_r.reshape(NW, EPT)
    ae2 = ae2_r.reshape(NW, EPT)
    ae_max = jnp.max(ae_m, axis=(0, 1))

    h1, a1, m1 = _node_mm(x, W1, jnp.stack([att_src1, att_dst1], axis=1))
    c1 = jnp.maximum(jnp.max(m1[..., 0]) + jnp.max(m1[..., 1]) + ae_max[0], 0.0)
    ex1, es1_p, sae1_p, deg_p = _edge_scalar_l1(
        a1.T, srcf, dstf, ae1, jnp.full((L,), c1, f32))
    num1_p, = _row_pass(h1, src3, dst3, ex1.reshape(NW, NCHK, CH), zeros)
    deg_t = deg_p.T
    out1 = _combine(num1_p, deg_t, sae1_p.T, es1_p.T, a1, h1,
                    c1.reshape(1, 1), b1[None, :])

    h2, a2, m2 = _node_mm(out1, W2, jnp.stack([att_src2, att_dst2], axis=1),
                          relu_in=True)
    c2 = jnp.maximum(jnp.max(m2[..., 0]) + jnp.max(m2[..., 1]) + ae_max[1], 0.0)
    ex2, es2_p, sae2_p = _edge_scalar_l2(a2.T, srcf, dstf, ae2,
                                         jnp.full((L,), c2, f32))
    num2_p, = _row_pass(h2, src3, dst3, ex2.reshape(NW, NCHK, CH), zeros)
    out2 = _combine(num2_p, deg_t, sae2_p.T, es2_p.T, a2, h2,
                    c2.reshape(1, 1), b2[None, :])
    return out2


# speedup vs baseline: 1.1365x; 1.1365x over previous
import jax
import jax.numpy as jnp
from jax import lax
from jax.experimental import pallas as pl
from jax.experimental.pallas import tpu as pltpu
from jax.experimental.pallas import tpu_sc as plsc

N = 10000
E = 320000
D = 128
DE = 16

NC = 2
NS = 16
NW = NC * NS
EPT = E // NW
CH = 80
NCHK = EPT // CH
L = 16
NPT = N // NS

_MESH = plsc.VectorSubcoreMesh(
    core_axis_name="c", subcore_axis_name="s", num_cores=NC, num_subcores=NS)
_SC_PARAMS = pltpu.CompilerParams(
    needs_layout_passes=False, use_tc_tiling_on_sc=False)

f32 = jnp.float32
i32 = jnp.int32


def _att2(h, att_ref):
    return lax.dot_general(att_ref[...], h, (((0,), (1,)), ((), ())),
                           preferred_element_type=f32)


def _node_mm(x, w, att):
    def body(x_ref, w_ref, att_ref, h_ref, a_ref, m_ref):
        h = jnp.dot(x_ref[...], w_ref[...], preferred_element_type=f32)
        h_ref[...] = h
        a = _att2(h, att_ref)
        a_ref[...] = a
        m_ref[0, :] = jnp.max(a, axis=1)

    return pl.pallas_call(
        body,
        out_shape=[
            jax.ShapeDtypeStruct((N, D), f32),
            jax.ShapeDtypeStruct((2, N), f32),
            jax.ShapeDtypeStruct((1, 2), f32),
        ],
    )(x, w, att)


def _edge_mv(flepe, w_edge1, att_edge1, w_edge2, att_edge2):
    def body(fl_ref, we1_ref, ate1_ref, we2_ref, ate2_ref,
             ae1_ref, ae2_ref, m_ref):
        i = pl.program_id(0)
        we1 = jnp.dot(we1_ref[...], ate1_ref[...], preferred_element_type=f32)
        we2 = jnp.dot(we2_ref[...], ate2_ref[...], preferred_element_type=f32)
        ae1 = jnp.dot(fl_ref[...], we1, preferred_element_type=f32)[:, 0]
        ae2 = jnp.dot(fl_ref[...], we2, preferred_element_type=f32)[:, 0]
        ae1_ref[0, 0, :] = ae1
        ae2_ref[0, 0, :] = ae2
        cur = jnp.stack([jnp.max(ae1), jnp.max(ae2)])

        @pl.when(i == 0)
        def _():
            m_ref[0, :] = cur

        @pl.when(i > 0)
        def _():
            m_ref[0, :] = jnp.maximum(m_ref[0, :], cur)

    return pl.pallas_call(
        body,
        grid=(NW,),
        in_specs=[
            pl.BlockSpec((EPT, DE), lambda i: (i, 0)),
            pl.BlockSpec((DE, D), lambda i: (0, 0)),
            pl.BlockSpec((D, 1), lambda i: (0, 0)),
            pl.BlockSpec((DE, D), lambda i: (0, 0)),
            pl.BlockSpec((D, 1), lambda i: (0, 0)),
        ],
        out_specs=[
            pl.BlockSpec((1, 1, EPT), lambda i: (i, 0, 0)),
            pl.BlockSpec((1, 1, EPT), lambda i: (i, 0, 0)),
            pl.BlockSpec((1, 2), lambda i: (0, 0)),
        ],
        out_shape=[
            jax.ShapeDtypeStruct((NW, 1, EPT), f32),
            jax.ShapeDtypeStruct((NW, 1, EPT), f32),
            jax.ShapeDtypeStruct((1, 2), f32),
        ],
    )(flepe, w_edge1, att_edge1[:, None], w_edge2, att_edge2[:, None])


def _make_edge_scalar(first):
    out_type = [jax.ShapeDtypeStruct((NW, EPT), f32),
                jax.ShapeDtypeStruct((NW, N), f32),
                jax.ShapeDtypeStruct((NW, N), f32)]
    scratch = [
        pltpu.VMEM((N,), f32),
        pltpu.VMEM((N,), f32),
        pltpu.VMEM((EPT,), i32),
        pltpu.VMEM((EPT,), i32),
        pltpu.VMEM((EPT,), f32),
        pltpu.VMEM((EPT,), f32),
        pltpu.VMEM((N,), f32),
        pltpu.VMEM((N,), f32),
        pltpu.VMEM((L,), f32),
    ]
    if first:
        out_type += [jax.ShapeDtypeStruct((NW, N), f32)]
        scratch += [pltpu.VMEM((N,), f32)]

    def body(asd_hbm, ei_hbm, ae_hbm, c_hbm, *rest):
        if first:
            (ex_hbm, es_hbm, sae_hbm, deg_hbm,
             asv, adv, srcv, dstv, aev, exv, esv, saev, cv, degv) = rest
        else:
            (ex_hbm, es_hbm, sae_hbm,
             asv, adv, srcv, dstv, aev, exv, esv, saev, cv) = rest
            degv = None
        wid = lax.axis_index("s") * NC + lax.axis_index("c")
        off = pl.multiple_of(wid * EPT, 8)
        pltpu.sync_copy(asd_hbm.at[0], asv)
        pltpu.sync_copy(asd_hbm.at[1], adv)
        pltpu.sync_copy(ei_hbm.at[0, pl.ds(off, EPT)], srcv)
        pltpu.sync_copy(ei_hbm.at[1, pl.ds(off, EPT)], dstv)
        pltpu.sync_copy(ae_hbm.at[wid, 0], aev)
        pltpu.sync_copy(c_hbm, cv)

        zz = jnp.zeros((L,), f32)

        def zbody(i, _):
            esv[pl.ds(i * L, L)] = zz
            saev[pl.ds(i * L, L)] = zz
            if first:
                degv[pl.ds(i * L, L)] = zz
            return 0

        lax.fori_loop(0, N // L, zbody, 0)

        cvec = cv[...]
        ones = jnp.full((L,), 1.0, f32)

        def ebody(i, _):
            sl = pl.ds(i * L, L)
            s = srcv[sl]
            d = dstv[sl]
            ae = aev[sl]
            z = plsc.load_gather(asv, [s]) + plsc.load_gather(adv, [d]) + ae
            alpha = jnp.maximum(z, 0.2 * z)
            ex = jnp.exp(alpha - cvec)
            exv[sl] = ex
            plsc.addupdate_scatter(esv, [d], ex)
            plsc.addupdate_scatter(saev, [d], ae)
            if first:
                plsc.addupdate_scatter(degv, [d], ones)
            return 0

        lax.fori_loop(0, EPT // L, ebody, 0)

        pltpu.sync_copy(exv, ex_hbm.at[wid])
        pltpu.sync_copy(esv, es_hbm.at[wid])
        pltpu.sync_copy(saev, sae_hbm.at[wid])
        if first:
            pltpu.sync_copy(degv, deg_hbm.at[wid])

    return pl.kernel(body, out_type=out_type, mesh=_MESH, scratch_types=scratch,
                     compiler_params=_SC_PARAMS,
                     name="edge_scalar_l1" if first else "edge_scalar_l2")


_edge_scalar_l1 = _make_edge_scalar(True)
_edge_scalar_l2 = _make_edge_scalar(False)


def _row_pass_body(h_hbm, ei_hbm, ex_hbm, z_hbm, num_hbm,
                   srcv, dstv, rows0, rows1, exb0, exb1, acc,
                   rsem0, rsem1, esem0, esem1):
    cid = lax.axis_index("c")
    sid = lax.axis_index("s")
    wid = sid * NC + cid
    off = pl.multiple_of(wid * EPT, 8)
    pltpu.sync_copy(ei_hbm.at[0, pl.ds(off, EPT)], srcv)
    pltpu.sync_copy(ei_hbm.at[1, pl.ds(off, EPT)], dstv)
    pltpu.sync_copy(z_hbm, acc.at[pl.ds(sid * NPT, NPT)])
    plsc.subcore_barrier()

    rows = (rows0, rows1)
    exb = (exb0, exb1)
    rsem = (rsem0, rsem1)
    esem = (esem0, esem1)

    def start(c, b):
        pltpu.async_copy(h_hbm.at[srcv.at[pl.ds(c * CH, CH)]], rows[b], rsem[b])
        pltpu.async_copy(ex_hbm.at[wid, pl.ds(c * CH, CH)], exb[b], esem[b])

    def finish(c, b):
        pltpu.make_async_copy(h_hbm.at[pl.ds(0, CH)], rows[b], rsem[b]).wait()
        pltpu.make_async_copy(ex_hbm.at[0, pl.ds(0, CH)], exb[b], esem[b]).wait()

        def scale(e, _):
            eidx = jnp.full((L,), 0, i32) + e
            s = plsc.load_gather(exb[b], [eidx])
            for k in range(D // L):
                sl = pl.ds(k * L, L)
                rows[b][e, sl] = rows[b][e, sl] * s
            return 0

        lax.fori_loop(0, CH, scale, 0)
        pltpu.sync_copy(rows[b], acc.at[dstv.at[pl.ds(c * CH, CH)]], add=True)

    start(0, 0)

    def gbody(g, _):
        c0 = 2 * g
        start(c0 + 1, 1)
        finish(c0, 0)

        @pl.when(g < NCHK // 2 - 1)
        def _():
            start(c0 + 2, 0)

        finish(c0 + 1, 1)
        return 0

    lax.fori_loop(0, NCHK // 2, gbody, 0)
    start(NCHK - 1, 0)
    finish(NCHK - 1, 0)

    plsc.subcore_barrier()
    pltpu.sync_copy(acc.at[pl.ds(sid * NPT, NPT)],
                    num_hbm.at[cid, pl.ds(sid * NPT, NPT)])


_row_pass = pl.kernel(
    _row_pass_body,
    out_type=[jax.ShapeDtypeStruct((NC, N, D), f32)],
    mesh=_MESH,
    scratch_types=[
        pltpu.VMEM((EPT,), i32),
        pltpu.VMEM((EPT,), i32),
        pltpu.VMEM((CH, D), f32),
        pltpu.VMEM((CH, D), f32),
        pltpu.VMEM((CH,), f32),
        pltpu.VMEM((CH,), f32),
        pltpu.VMEM_SHARED((N, D), f32),
        pltpu.SemaphoreType.DMA,
        pltpu.SemaphoreType.DMA,
        pltpu.SemaphoreType.DMA,
        pltpu.SemaphoreType.DMA,
    ],
    compiler_params=_SC_PARAMS,
    name="row_pass",
)


def _combine_core(nump_ref, degp_ref, saep_ref, esp_ref, a_ref, h_ref, c_ref,
                  b_ref):
    deg = jnp.sum(degp_ref[...], axis=0)
    sae = jnp.sum(saep_ref[...], axis=0)
    es = jnp.sum(esp_ref[...], axis=0)
    z = a_ref[0, :] + a_ref[1, :] + sae / jnp.maximum(deg, 1.0)
    alpha = jnp.maximum(z, 0.2 * z)
    selfw = jnp.exp(alpha - c_ref[0, 0])
    denom = es + selfw
    num = nump_ref[0] + nump_ref[1] + selfw[:, None] * h_ref[...]
    return num / denom[:, None] + b_ref[...]


def _combine_mm(num_p, deg_p, sae_p, es_p, a, h, c, b, w, att):
    def body(nump_ref, degp_ref, saep_ref, esp_ref, a_ref, h_ref, c_ref,
             b_ref, w_ref, att_ref, h2_ref, a2_ref, m_ref):
        out = _combine_core(nump_ref, degp_ref, saep_ref, esp_ref, a_ref,
                            h_ref, c_ref, b_ref)
        h2 = jnp.dot(jax.nn.relu(out), w_ref[...], preferred_element_type=f32)
        h2_ref[...] = h2
        a2 = _att2(h2, att_ref)
        a2_ref[...] = a2
        m_ref[0, :] = jnp.max(a2, axis=1)

    return pl.pallas_call(
        body,
        out_shape=[
            jax.ShapeDtypeStruct((N, D), f32),
            jax.ShapeDtypeStruct((2, N), f32),
            jax.ShapeDtypeStruct((1, 2), f32),
        ],
    )(num_p, deg_p, sae_p, es_p, a, h, c, b, w, att)


def _combine_final(num_p, deg_p, sae_p, es_p, a, h, c, b):
    def body(nump_ref, degp_ref, saep_ref, esp_ref, a_ref, h_ref, c_ref,
             b_ref, out_ref):
        out_ref[...] = _combine_core(nump_ref, degp_ref, saep_ref, esp_ref,
                                     a_ref, h_ref, c_ref, b_ref)

    return pl.pallas_call(
        body,
        out_shape=jax.ShapeDtypeStruct((N, D), f32),
    )(num_p, deg_p, sae_p, es_p, a, h, c, b)


def kernel(x, edge_index, flepe, W1, att_src1, att_dst1, W_edge1, att_edge1, b1,
           W2, att_src2, att_dst2, W_edge2, att_edge2, b2):
    zeros = jnp.zeros((NPT, D), f32)

    ae1, ae2, ae_m = _edge_mv(flepe, W_edge1, att_edge1, W_edge2, att_edge2)
    ae_max = ae_m[0]

    h1, a1, m1 = _node_mm(x, W1, jnp.stack([att_src1, att_dst1], axis=1))
    c1 = jnp.maximum(m1[0, 0] + m1[0, 1] + ae_max[0], 0.0)
    ex1, es1_p, sae1_p, deg_p = _edge_scalar_l1(
        a1, edge_index, ae1, jnp.full((L,), c1, f32))
    num1_p, = _row_pass(h1, edge_index, ex1, zeros)
    h2, a2, m2 = _combine_mm(num1_p, deg_p, sae1_p, es1_p, a1, h1,
                             c1.reshape(1, 1), b1[None, :], W2,
                             jnp.stack([att_src2, att_dst2], axis=1))

    c2 = jnp.maximum(m2[0, 0] + m2[0, 1] + ae_max[1], 0.0)
    ex2, es2_p, sae2_p = _edge_scalar_l2(
        a2, edge_index, ae2, jnp.full((L,), c2, f32))
    num2_p, = _row_pass(h2, edge_index, ex2, zeros)
    return _combine_final(num2_p, deg_p, sae2_p, es2_p, a2, h2,
                          c2.reshape(1, 1), b2[None, :])


# speedup vs baseline: 1.5171x; 1.3349x over previous
import jax
import jax.numpy as jnp
from jax import lax
from jax.experimental import pallas as pl
from jax.experimental.pallas import tpu as pltpu
from jax.experimental.pallas import tpu_sc as plsc

N = 10000
E = 320000
D = 128
DE = 16

NC = 2
NS = 16
NW = NC * NS
EPT = E // NW
CH = 80
NCHK = EPT // CH
L = 16
NPT = N // NS

_MESH = plsc.VectorSubcoreMesh(
    core_axis_name="c", subcore_axis_name="s", num_cores=NC, num_subcores=NS)
_SC_PARAMS = pltpu.CompilerParams(
    needs_layout_passes=False, use_tc_tiling_on_sc=False)

f32 = jnp.float32
i32 = jnp.int32


def _att2(h, att_ref):
    return lax.dot_general(att_ref[...], h, (((0,), (1,)), ((), ())),
                           preferred_element_type=f32)


def _node_mm(x, w, att):
    def body(x_ref, w_ref, att_ref, h_ref, a_ref, m_ref):
        h = jnp.dot(x_ref[...], w_ref[...], preferred_element_type=f32)
        h_ref[...] = h
        a = _att2(h, att_ref)
        a_ref[...] = a
        m_ref[0, :] = jnp.max(a, axis=1)

    return pl.pallas_call(
        body,
        out_shape=[
            jax.ShapeDtypeStruct((N, D), f32),
            jax.ShapeDtypeStruct((2, N), f32),
            jax.ShapeDtypeStruct((1, 2), f32),
        ],
    )(x, w, att)


def _edge_mv(flr, bmat):
    RPB = EPT // 8

    def body(fl_ref, b_ref, ae_ref, m_ref):
        i = pl.program_id(0)
        a = jnp.dot(fl_ref[0], b_ref[...], preferred_element_type=f32)
        ae_ref[0] = a
        cur = jnp.stack([jnp.max(a[:, 0:8]), jnp.max(a[:, 8:16])])

        @pl.when(i == 0)
        def _():
            m_ref[0, :] = cur

        @pl.when(i > 0)
        def _():
            m_ref[0, :] = jnp.maximum(m_ref[0, :], cur)

    return pl.pallas_call(
        body,
        grid=(NW,),
        in_specs=[
            pl.BlockSpec((1, RPB, D), lambda i: (i, 0, 0)),
            pl.BlockSpec((D, DE), lambda i: (0, 0)),
        ],
        out_specs=[
            pl.BlockSpec((1, RPB, DE), lambda i: (i, 0, 0)),
            pl.BlockSpec((1, 2), lambda i: (0, 0)),
        ],
        out_shape=[
            jax.ShapeDtypeStruct((NW, RPB, DE), f32),
            jax.ShapeDtypeStruct((1, 2), f32),
        ],
    )(flr, bmat)


def _make_edge_scalar(first):
    out_type = [jax.ShapeDtypeStruct((NW, EPT), f32),
                jax.ShapeDtypeStruct((NW, N), f32),
                jax.ShapeDtypeStruct((NW, N), f32)]
    scratch = [
        pltpu.VMEM((N,), f32),
        pltpu.VMEM((N,), f32),
        pltpu.VMEM((EPT,), i32),
        pltpu.VMEM((EPT,), i32),
        pltpu.VMEM((EPT // 8, DE), f32),
        pltpu.VMEM((EPT,), f32),
        pltpu.VMEM((N,), f32),
        pltpu.VMEM((N,), f32),
        pltpu.VMEM((L,), f32),
    ]
    if first:
        out_type += [jax.ShapeDtypeStruct((NW, N), f32)]
        scratch += [pltpu.VMEM((N,), f32)]

    def body(asd_hbm, ei_hbm, ae_hbm, c_hbm, *rest):
        if first:
            (ex_hbm, es_hbm, sae_hbm, deg_hbm,
             asv, adv, srcv, dstv, aev, exv, esv, saev, cv, degv) = rest
        else:
            (ex_hbm, es_hbm, sae_hbm,
             asv, adv, srcv, dstv, aev, exv, esv, saev, cv) = rest
            degv = None
        wid = lax.axis_index("s") * NC + lax.axis_index("c")
        off = pl.multiple_of(wid * EPT, 8)
        pltpu.sync_copy(asd_hbm.at[0], asv)
        pltpu.sync_copy(asd_hbm.at[1], adv)
        pltpu.sync_copy(ei_hbm.at[0, pl.ds(off, EPT)], srcv)
        pltpu.sync_copy(ei_hbm.at[1, pl.ds(off, EPT)], dstv)
        pltpu.sync_copy(ae_hbm.at[wid], aev)
        pltpu.sync_copy(c_hbm, cv)

        zz = jnp.zeros((L,), f32)

        def zbody(i, _):
            esv[pl.ds(i * L, L)] = zz
            saev[pl.ds(i * L, L)] = zz
            if first:
                degv[pl.ds(i * L, L)] = zz
            return 0

        lax.fori_loop(0, N // L, zbody, 0)

        cvec = cv[...]
        ones = jnp.full((L,), 1.0, f32)
        lane = lax.iota(i32, L)
        rofs = lane // 8
        cofs = (lane % 8) + (0 if first else 8)

        def ebody(i, _):
            sl = pl.ds(i * L, L)
            s = srcv[sl]
            d = dstv[sl]
            ae = plsc.load_gather(aev, [2 * i + rofs, cofs])
            z = plsc.load_gather(asv, [s]) + plsc.load_gather(adv, [d]) + ae
            alpha = jnp.maximum(z, 0.2 * z)
            ex = jnp.exp(alpha - cvec)
            exv[sl] = ex
            plsc.addupdate_scatter(esv, [d], ex)
            plsc.addupdate_scatter(saev, [d], ae)
            if first:
                plsc.addupdate_scatter(degv, [d], ones)
            return 0

        lax.fori_loop(0, EPT // L, ebody, 0)

        pltpu.sync_copy(exv, ex_hbm.at[wid])
        pltpu.sync_copy(esv, es_hbm.at[wid])
        pltpu.sync_copy(saev, sae_hbm.at[wid])
        if first:
            pltpu.sync_copy(degv, deg_hbm.at[wid])

    return pl.kernel(body, out_type=out_type, mesh=_MESH, scratch_types=scratch,
                     compiler_params=_SC_PARAMS,
                     name="edge_scalar_l1" if first else "edge_scalar_l2")


_edge_scalar_l1 = _make_edge_scalar(True)
_edge_scalar_l2 = _make_edge_scalar(False)


def _row_pass_body(h_hbm, ei_hbm, ex_hbm, z_hbm, num_hbm,
                   srcv, dstv, rows0, rows1, exb0, exb1, acc,
                   rsem0, rsem1, esem0, esem1):
    cid = lax.axis_index("c")
    sid = lax.axis_index("s")
    wid = sid * NC + cid
    off = pl.multiple_of(wid * EPT, 8)
    pltpu.sync_copy(ei_hbm.at[0, pl.ds(off, EPT)], srcv)
    pltpu.sync_copy(ei_hbm.at[1, pl.ds(off, EPT)], dstv)
    pltpu.sync_copy(z_hbm, acc.at[pl.ds(sid * NPT, NPT)])
    plsc.subcore_barrier()

    rows = (rows0, rows1)
    exb = (exb0, exb1)
    rsem = (rsem0, rsem1)
    esem = (esem0, esem1)

    def start(c, b):
        pltpu.async_copy(h_hbm.at[srcv.at[pl.ds(c * CH, CH)]], rows[b], rsem[b])
        pltpu.async_copy(ex_hbm.at[wid, pl.ds(c * CH, CH)], exb[b], esem[b])

    def finish(c, b):
        pltpu.make_async_copy(h_hbm.at[pl.ds(0, CH)], rows[b], rsem[b]).wait()
        pltpu.make_async_copy(ex_hbm.at[0, pl.ds(0, CH)], exb[b], esem[b]).wait()

        def scale(e, _):
            eidx = jnp.full((L,), 0, i32) + e
            s = plsc.load_gather(exb[b], [eidx])
            for k in range(D // L):
                sl = pl.ds(k * L, L)
                rows[b][e, sl] = rows[b][e, sl] * s
            return 0

        lax.fori_loop(0, CH, scale, 0)
        pltpu.sync_copy(rows[b], acc.at[dstv.at[pl.ds(c * CH, CH)]], add=True)

    start(0, 0)

    def gbody(g, _):
        c0 = 2 * g
        start(c0 + 1, 1)
        finish(c0, 0)

        @pl.when(g < NCHK // 2 - 1)
        def _():
            start(c0 + 2, 0)

        finish(c0 + 1, 1)
        return 0

    lax.fori_loop(0, NCHK // 2, gbody, 0)
    start(NCHK - 1, 0)
    finish(NCHK - 1, 0)

    plsc.subcore_barrier()
    pltpu.sync_copy(acc.at[pl.ds(sid * NPT, NPT)],
                    num_hbm.at[cid, pl.ds(sid * NPT, NPT)])


_row_pass = pl.kernel(
    _row_pass_body,
    out_type=[jax.ShapeDtypeStruct((NC, N, D), f32)],
    mesh=_MESH,
    scratch_types=[
        pltpu.VMEM((EPT,), i32),
        pltpu.VMEM((EPT,), i32),
        pltpu.VMEM((CH, D), f32),
        pltpu.VMEM((CH, D), f32),
        pltpu.VMEM((CH,), f32),
        pltpu.VMEM((CH,), f32),
        pltpu.VMEM_SHARED((N, D), f32),
        pltpu.SemaphoreType.DMA,
        pltpu.SemaphoreType.DMA,
        pltpu.SemaphoreType.DMA,
        pltpu.SemaphoreType.DMA,
    ],
    compiler_params=_SC_PARAMS,
    name="row_pass",
)


def _combine_core(nump_ref, degp_ref, saep_ref, esp_ref, a_ref, h_ref, c_ref,
                  b_ref):
    deg = jnp.sum(degp_ref[...], axis=0)
    sae = jnp.sum(saep_ref[...], axis=0)
    es = jnp.sum(esp_ref[...], axis=0)
    z = a_ref[0, :] + a_ref[1, :] + sae / jnp.maximum(deg, 1.0)
    alpha = jnp.maximum(z, 0.2 * z)
    selfw = jnp.exp(alpha - c_ref[0, 0])
    denom = es + selfw
    num = nump_ref[0] + nump_ref[1] + selfw[:, None] * h_ref[...]
    return num / denom[:, None] + b_ref[...]


def _combine_mm(num_p, deg_p, sae_p, es_p, a, h, c, b, w, att):
    def body(nump_ref, degp_ref, saep_ref, esp_ref, a_ref, h_ref, c_ref,
             b_ref, w_ref, att_ref, h2_ref, a2_ref, m_ref):
        out = _combine_core(nump_ref, degp_ref, saep_ref, esp_ref, a_ref,
                            h_ref, c_ref, b_ref)
        h2 = jnp.dot(jax.nn.relu(out), w_ref[...], preferred_element_type=f32)
        h2_ref[...] = h2
        a2 = _att2(h2, att_ref)
        a2_ref[...] = a2
        m_ref[0, :] = jnp.max(a2, axis=1)

    return pl.pallas_call(
        body,
        out_shape=[
            jax.ShapeDtypeStruct((N, D), f32),
            jax.ShapeDtypeStruct((2, N), f32),
            jax.ShapeDtypeStruct((1, 2), f32),
        ],
    )(num_p, deg_p, sae_p, es_p, a, h, c, b, w, att)


def _combine_final(num_p, deg_p, sae_p, es_p, a, h, c, b):
    def body(nump_ref, degp_ref, saep_ref, esp_ref, a_ref, h_ref, c_ref,
             b_ref, out_ref):
        out_ref[...] = _combine_core(nump_ref, degp_ref, saep_ref, esp_ref,
                                     a_ref, h_ref, c_ref, b_ref)

    return pl.pallas_call(
        body,
        out_shape=jax.ShapeDtypeStruct((N, D), f32),
    )(num_p, deg_p, sae_p, es_p, a, h, c, b)


def kernel(x, edge_index, flepe, W1, att_src1, att_dst1, W_edge1, att_edge1, b1,
           W2, att_src2, att_dst2, W_edge2, att_edge2, b2):
    zeros = jnp.zeros((NPT, D), f32)

    eye8 = jnp.eye(8, dtype=f32)
    bmat = jnp.concatenate(
        [jnp.kron(eye8, (W_edge1 @ att_edge1)[:, None]),
         jnp.kron(eye8, (W_edge2 @ att_edge2)[:, None])], axis=1)
    ae, ae_m = _edge_mv(flepe.reshape(NW, EPT // 8, D), bmat)
    ae_max = ae_m[0]

    h1, a1, m1 = _node_mm(x, W1, jnp.stack([att_src1, att_dst1], axis=1))
    c1 = jnp.maximum(m1[0, 0] + m1[0, 1] + ae_max[0], 0.0)
    ex1, es1_p, sae1_p, deg_p = _edge_scalar_l1(
        a1, edge_index, ae, jnp.full((L,), c1, f32))
    num1_p, = _row_pass(h1, edge_index, ex1, zeros)
    h2, a2, m2 = _combine_mm(num1_p, deg_p, sae1_p, es1_p, a1, h1,
                             c1.reshape(1, 1), b1[None, :], W2,
                             jnp.stack([att_src2, att_dst2], axis=1))

    c2 = jnp.maximum(m2[0, 0] + m2[0, 1] + ae_max[1], 0.0)
    ex2, es2_p, sae2_p = _edge_scalar_l2(
        a2, edge_index, ae, jnp.full((L,), c2, f32))
    num2_p, = _row_pass(h2, edge_index, ex2, zeros)
    return _combine_final(num2_p, deg_p, sae2_p, es2_p, a2, h2,
                          c2.reshape(1, 1), b2[None, :])


# speedup vs baseline: 1.6403x; 1.0812x over previous
import jax
import jax.numpy as jnp
from jax import lax
from jax.experimental import pallas as pl
from jax.experimental.pallas import tpu as pltpu
from jax.experimental.pallas import tpu_sc as plsc

N = 10000
E = 320000
D = 128
DE = 16

NC = 2
NS = 16
NW = NC * NS
EPT = E // NW
CH = 80
NCHK = EPT // CH
L = 16
NPT = N // NS

_MESH = plsc.VectorSubcoreMesh(
    core_axis_name="c", subcore_axis_name="s", num_cores=NC, num_subcores=NS)
_SC_PARAMS = pltpu.CompilerParams(
    needs_layout_passes=False, use_tc_tiling_on_sc=False)

f32 = jnp.float32
i32 = jnp.int32


def _att2(h, att_ref):
    return lax.dot_general(att_ref[...], h, (((0,), (1,)), ((), ())),
                           preferred_element_type=f32)


def _node_mm(x, w, att):
    def body(x_ref, w_ref, att_ref, h_ref, a_ref, m_ref):
        h = jnp.dot(x_ref[...], w_ref[...], preferred_element_type=f32)
        h_ref[...] = h
        a = _att2(h, att_ref)
        a_ref[...] = a
        m_ref[0, :] = jnp.max(a, axis=1)

    return pl.pallas_call(
        body,
        out_shape=[
            jax.ShapeDtypeStruct((N, D), f32),
            jax.ShapeDtypeStruct((2, N), f32),
            jax.ShapeDtypeStruct((1, 2), f32),
        ],
    )(x, w, att)


def _edge_mv(flr, bmat):
    B = 1000

    def body(fl_ref, b_ref, ae_ref, m_ref):
        i = pl.program_id(0)
        a = jnp.dot(fl_ref[...], b_ref[...], preferred_element_type=f32)
        ae_ref[...] = a
        cur = jnp.stack([jnp.max(a[:, 0:8]), jnp.max(a[:, 8:16])])

        @pl.when(i == 0)
        def _():
            m_ref[0, :] = cur

        @pl.when(i > 0)
        def _():
            m_ref[0, :] = jnp.maximum(m_ref[0, :], cur)

    return pl.pallas_call(
        body,
        grid=(E // 8 // B,),
        in_specs=[
            pl.BlockSpec((B, D), lambda i: (i, 0)),
            pl.BlockSpec((D, DE), lambda i: (0, 0)),
        ],
        out_specs=[
            pl.BlockSpec((B, DE), lambda i: (i, 0)),
            pl.BlockSpec((1, 2), lambda i: (0, 0)),
        ],
        out_shape=[
            jax.ShapeDtypeStruct((E // 8, DE), f32),
            jax.ShapeDtypeStruct((1, 2), f32),
        ],
    )(flr, bmat)


def _make_edge_scalar(first):
    out_type = [jax.ShapeDtypeStruct((NW, EPT), f32),
                jax.ShapeDtypeStruct((NW, N), f32),
                jax.ShapeDtypeStruct((NW, N), f32)]
    scratch = [
        pltpu.VMEM((N,), f32),
        pltpu.VMEM((N,), f32),
        pltpu.VMEM((EPT,), i32),
        pltpu.VMEM((EPT,), i32),
        pltpu.VMEM((EPT // 8, DE), f32),
        pltpu.VMEM((EPT,), f32),
        pltpu.VMEM((N,), f32),
        pltpu.VMEM((N,), f32),
        pltpu.VMEM((L,), f32),
    ]
    if first:
        out_type += [jax.ShapeDtypeStruct((NW, N), f32)]
        scratch += [pltpu.VMEM((N,), f32)]

    def body(asd_hbm, ei_hbm, ae_hbm, c_hbm, *rest):
        if first:
            (ex_hbm, es_hbm, sae_hbm, deg_hbm,
             asv, adv, srcv, dstv, aev, exv, esv, saev, cv, degv) = rest
        else:
            (ex_hbm, es_hbm, sae_hbm,
             asv, adv, srcv, dstv, aev, exv, esv, saev, cv) = rest
            degv = None
        wid = lax.axis_index("s") * NC + lax.axis_index("c")
        off = pl.multiple_of(wid * EPT, 8)
        pltpu.sync_copy(asd_hbm.at[0], asv)
        pltpu.sync_copy(asd_hbm.at[1], adv)
        pltpu.sync_copy(ei_hbm.at[0, pl.ds(off, EPT)], srcv)
        pltpu.sync_copy(ei_hbm.at[1, pl.ds(off, EPT)], dstv)
        pltpu.sync_copy(ae_hbm.at[pl.ds(wid * (EPT // 8), EPT // 8)], aev)
        pltpu.sync_copy(c_hbm, cv)

        zz = jnp.zeros((L,), f32)

        def zbody(i, _):
            esv[pl.ds(i * L, L)] = zz
            saev[pl.ds(i * L, L)] = zz
            if first:
                degv[pl.ds(i * L, L)] = zz
            return 0

        lax.fori_loop(0, N // L, zbody, 0)

        cvec = cv[...]
        ones = jnp.full((L,), 1.0, f32)
        lane = lax.iota(i32, L)
        rofs = lane // 8
        cofs = (lane % 8) + (0 if first else 8)

        def ebody(i, _):
            sl = pl.ds(i * L, L)
            s = srcv[sl]
            d = dstv[sl]
            ae = plsc.load_gather(aev, [2 * i + rofs, cofs])
            z = plsc.load_gather(asv, [s]) + plsc.load_gather(adv, [d]) + ae
            alpha = jnp.maximum(z, 0.2 * z)
            ex = jnp.exp(alpha - cvec)
            exv[sl] = ex
            plsc.addupdate_scatter(esv, [d], ex)
            plsc.addupdate_scatter(saev, [d], ae)
            if first:
                plsc.addupdate_scatter(degv, [d], ones)
            return 0

        lax.fori_loop(0, EPT // L, ebody, 0)

        pltpu.sync_copy(exv, ex_hbm.at[wid])
        pltpu.sync_copy(esv, es_hbm.at[wid])
        pltpu.sync_copy(saev, sae_hbm.at[wid])
        if first:
            pltpu.sync_copy(degv, deg_hbm.at[wid])

    return pl.kernel(body, out_type=out_type, mesh=_MESH, scratch_types=scratch,
                     compiler_params=_SC_PARAMS,
                     name="edge_scalar_l1" if first else "edge_scalar_l2")


_edge_scalar_l1 = _make_edge_scalar(True)
_edge_scalar_l2 = _make_edge_scalar(False)


def _row_pass_body(h_hbm, ei_hbm, ex_hbm, z_hbm, num_hbm,
                   srcv, dstv, rows0, rows1, exb0, exb1, acc,
                   rsem0, rsem1, esem0, esem1):
    cid = lax.axis_index("c")
    sid = lax.axis_index("s")
    wid = sid * NC + cid
    off = pl.multiple_of(wid * EPT, 8)
    pltpu.sync_copy(ei_hbm.at[0, pl.ds(off, EPT)], srcv)
    pltpu.sync_copy(ei_hbm.at[1, pl.ds(off, EPT)], dstv)
    pltpu.sync_copy(z_hbm, acc.at[pl.ds(sid * NPT, NPT)])
    plsc.subcore_barrier()

    rows = (rows0, rows1)
    exb = (exb0, exb1)
    rsem = (rsem0, rsem1)
    esem = (esem0, esem1)

    def start(c, b):
        pltpu.async_copy(h_hbm.at[srcv.at[pl.ds(c * CH, CH)]], rows[b], rsem[b])
        pltpu.async_copy(ex_hbm.at[wid, pl.ds(c * CH, CH)], exb[b], esem[b])

    def finish(c, b):
        pltpu.make_async_copy(h_hbm.at[pl.ds(0, CH)], rows[b], rsem[b]).wait()
        pltpu.make_async_copy(ex_hbm.at[0, pl.ds(0, CH)], exb[b], esem[b]).wait()

        def scale(e, _):
            eidx = jnp.full((L,), 0, i32) + e
            s = plsc.load_gather(exb[b], [eidx])
            for k in range(D // L):
                sl = pl.ds(k * L, L)
                rows[b][e, sl] = rows[b][e, sl] * s
            return 0

        lax.fori_loop(0, CH, scale, 0)
        pltpu.sync_copy(rows[b], acc.at[dstv.at[pl.ds(c * CH, CH)]], add=True)

    start(0, 0)

    def gbody(g, _):
        c0 = 2 * g
        start(c0 + 1, 1)
        finish(c0, 0)

        @pl.when(g < NCHK // 2 - 1)
        def _():
            start(c0 + 2, 0)

        finish(c0 + 1, 1)
        return 0

    lax.fori_loop(0, NCHK // 2, gbody, 0)
    start(NCHK - 1, 0)
    finish(NCHK - 1, 0)

    plsc.subcore_barrier()
    pltpu.sync_copy(acc.at[pl.ds(sid * NPT, NPT)],
                    num_hbm.at[cid, pl.ds(sid * NPT, NPT)])


_row_pass = pl.kernel(
    _row_pass_body,
    out_type=[jax.ShapeDtypeStruct((NC, N, D), f32)],
    mesh=_MESH,
    scratch_types=[
        pltpu.VMEM((EPT,), i32),
        pltpu.VMEM((EPT,), i32),
        pltpu.VMEM((CH, D), f32),
        pltpu.VMEM((CH, D), f32),
        pltpu.VMEM((CH,), f32),
        pltpu.VMEM((CH,), f32),
        pltpu.VMEM_SHARED((N, D), f32),
        pltpu.SemaphoreType.DMA,
        pltpu.SemaphoreType.DMA,
        pltpu.SemaphoreType.DMA,
        pltpu.SemaphoreType.DMA,
    ],
    compiler_params=_SC_PARAMS,
    name="row_pass",
)


def _combine_core(nump_ref, degp_ref, saep_ref, esp_ref, a_ref, h_ref, c_ref,
                  b_ref):
    deg = jnp.sum(degp_ref[...], axis=0)
    sae = jnp.sum(saep_ref[...], axis=0)
    es = jnp.sum(esp_ref[...], axis=0)
    z = a_ref[0, :] + a_ref[1, :] + sae / jnp.maximum(deg, 1.0)
    alpha = jnp.maximum(z, 0.2 * z)
    selfw = jnp.exp(alpha - c_ref[0, 0])
    denom = es + selfw
    num = nump_ref[0] + nump_ref[1] + selfw[:, None] * h_ref[...]
    return num / denom[:, None] + b_ref[...]


def _combine_mm(num_p, deg_p, sae_p, es_p, a, h, c, b, w, att):
    def body(nump_ref, degp_ref, saep_ref, esp_ref, a_ref, h_ref, c_ref,
             b_ref, w_ref, att_ref, h2_ref, a2_ref, m_ref):
        out = _combine_core(nump_ref, degp_ref, saep_ref, esp_ref, a_ref,
                            h_ref, c_ref, b_ref)
        h2 = jnp.dot(jax.nn.relu(out), w_ref[...], preferred_element_type=f32)
        h2_ref[...] = h2
        a2 = _att2(h2, att_ref)
        a2_ref[...] = a2
        m_ref[0, :] = jnp.max(a2, axis=1)

    return pl.pallas_call(
        body,
        out_shape=[
            jax.ShapeDtypeStruct((N, D), f32),
            jax.ShapeDtypeStruct((2, N), f32),
            jax.ShapeDtypeStruct((1, 2), f32),
        ],
    )(num_p, deg_p, sae_p, es_p, a, h, c, b, w, att)


def _combine_final(num_p, deg_p, sae_p, es_p, a, h, c, b):
    def body(nump_ref, degp_ref, saep_ref, esp_ref, a_ref, h_ref, c_ref,
             b_ref, out_ref):
        out_ref[...] = _combine_core(nump_ref, degp_ref, saep_ref, esp_ref,
                                     a_ref, h_ref, c_ref, b_ref)

    return pl.pallas_call(
        body,
        out_shape=jax.ShapeDtypeStruct((N, D), f32),
    )(num_p, deg_p, sae_p, es_p, a, h, c, b)


def kernel(x, edge_index, flepe, W1, att_src1, att_dst1, W_edge1, att_edge1, b1,
           W2, att_src2, att_dst2, W_edge2, att_edge2, b2):
    zeros = jnp.zeros((NPT, D), f32)

    eye8 = jnp.eye(8, dtype=f32)
    bmat = jnp.concatenate(
        [jnp.kron(eye8, (W_edge1 @ att_edge1)[:, None]),
         jnp.kron(eye8, (W_edge2 @ att_edge2)[:, None])], axis=1)
    ae, ae_m = _edge_mv(flepe.reshape(E // 8, D), bmat)
    ae_max = ae_m[0]

    h1, a1, m1 = _node_mm(x, W1, jnp.stack([att_src1, att_dst1], axis=1))
    c1 = jnp.maximum(m1[0, 0] + m1[0, 1] + ae_max[0], 0.0)
    ex1, es1_p, sae1_p, deg_p = _edge_scalar_l1(
        a1, edge_index, ae, jnp.full((L,), c1, f32))
    num1_p, = _row_pass(h1, edge_index, ex1, zeros)
    h2, a2, m2 = _combine_mm(num1_p, deg_p, sae1_p, es1_p, a1, h1,
                             c1.reshape(1, 1), b1[None, :], W2,
                             jnp.stack([att_src2, att_dst2], axis=1))

    c2 = jnp.maximum(m2[0, 0] + m2[0, 1] + ae_max[1], 0.0)
    ex2, es2_p, sae2_p = _edge_scalar_l2(
        a2, edge_index, ae, jnp.full((L,), c2, f32))
    num2_p, = _row_pass(h2, edge_index, ex2, zeros)
    return _combine_final(num2_p, deg_p, sae2_p, es2_p, a2, h2,
                          c2.reshape(1, 1), b2[None, :])
